# baseline (speedup 1.0000x reference)
import functools

import jax
import jax.numpy as jnp
from jax import lax
from jax.experimental import pallas as pl
from jax.experimental.pallas import tpu as pltpu

H = 256
HH = 128
ROW_BLK = 1000
EDGE_BLK = 2000


def _row(i):
    return (i, 0)


def _fixed(i):
    return (0, 0)


def _stats_update(pe):
    s0 = jnp.sum(pe, axis=0, keepdims=True)
    s1 = jnp.sum(pe * pe, axis=0, keepdims=True)
    return jnp.concatenate([s0, s1, jnp.zeros((6, H), jnp.float32)], axis=0)


def _enc2_body(x, w1, b1, w2lo, w2hi, b2lo, b2hi, olo, ohi):
    h = jax.nn.relu(x[...] * w1[...] + b1[...])
    olo[...] = jnp.dot(h, w2lo[...], preferred_element_type=jnp.float32) + b2lo[...]
    ohi[...] = jnp.dot(h, w2hi[...], preferred_element_type=jnp.float32) + b2hi[...]


def _enc2(x, p):
    n = x.shape[0]
    w2 = p["l2"]["W"]
    b2 = p["l2"]["b"]
    out = pl.pallas_call(
        _enc2_body,
        grid=(n // ROW_BLK,),
        in_specs=[pl.BlockSpec((ROW_BLK, 1), _row),
                  pl.BlockSpec((1, H), _fixed), pl.BlockSpec((1, H), _fixed),
                  pl.BlockSpec((H, HH), _fixed), pl.BlockSpec((H, HH), _fixed),
                  pl.BlockSpec((1, HH), _fixed), pl.BlockSpec((1, HH), _fixed)],
        out_specs=[pl.BlockSpec((ROW_BLK, HH), _row)] * 2,
        out_shape=[jax.ShapeDtypeStruct((n, HH), jnp.float32)] * 2,
    )(x, p["l1"]["W"].reshape(1, H), p["l1"]["b"].reshape(1, H),
      w2[:, :HH], w2[:, HH:], b2[:HH].reshape(1, HH), b2[HH:].reshape(1, HH))
    return out


def _edge_mlp_body(a, w1, b1, w2lo, w2hi, b2lo, b2hi, plo, phi, stats):
    i = pl.program_id(0)
    h = jax.nn.relu(a[...] * w1[...] + b1[...])
    pl_ = jax.nn.relu(jnp.dot(h, w2lo[...], preferred_element_type=jnp.float32) + b2lo[...])
    ph_ = jax.nn.relu(jnp.dot(h, w2hi[...], preferred_element_type=jnp.float32) + b2hi[...])
    plo[...] = pl_
    phi[...] = ph_

    @pl.when(i == 0)
    def _():
        stats[...] = jnp.zeros((8, H), jnp.float32)

    pe = jnp.concatenate([pl_, ph_], axis=1)
    stats[...] += _stats_update(pe)


def _edge_mlp(a, p):
    e = a.shape[0]
    w2 = p["l2"]["W"]
    b2 = p["l2"]["b"]
    plo, phi, stats = pl.pallas_call(
        _edge_mlp_body,
        grid=(e // EDGE_BLK,),
        in_specs=[pl.BlockSpec((EDGE_BLK, 1), _row),
                  pl.BlockSpec((1, H), _fixed), pl.BlockSpec((1, H), _fixed),
                  pl.BlockSpec((H, HH), _fixed), pl.BlockSpec((H, HH), _fixed),
                  pl.BlockSpec((1, HH), _fixed), pl.BlockSpec((1, HH), _fixed)],
        out_specs=[pl.BlockSpec((EDGE_BLK, HH), _row),
                   pl.BlockSpec((EDGE_BLK, HH), _row),
                   pl.BlockSpec((8, H), _fixed)],
        out_shape=[jax.ShapeDtypeStruct((e, HH), jnp.float32),
                   jax.ShapeDtypeStruct((e, HH), jnp.float32),
                   jax.ShapeDtypeStruct((8, H), jnp.float32)],
    )(a, p["l1"]["W"].reshape(1, H), p["l1"]["b"].reshape(1, H),
      w2[:, :HH], w2[:, HH:], b2[:HH].reshape(1, HH), b2[HH:].reshape(1, HH))
    return plo, phi, stats


def _bn_affine(stats, n, g, be):
    mean = stats[0] / n
    var = stats[1] / n - mean * mean
    s = g / jnp.sqrt(var + 1e-5)
    t = be - mean * s
    return s, t


def _node_mlp_body(xlo, xhi, alo, ahi, cnt, eps, w1lo, w1hi, b1, w2, b2,
                   out, stats):
    i = pl.program_id(0)
    inv = 1.0 / jnp.maximum(cnt[...], 1.0)
    e1 = 1.0 + eps[0, 0]
    zlo = e1 * xlo[...] + alo[...] * inv
    zhi = e1 * xhi[...] + ahi[...] * inv
    h = jax.nn.relu(jnp.dot(zlo, w1lo[...], preferred_element_type=jnp.float32)
                    + jnp.dot(zhi, w1hi[...], preferred_element_type=jnp.float32)
                    + b1[...])
    pe = jax.nn.relu(jnp.dot(h, w2[...], preferred_element_type=jnp.float32) + b2[...])
    out[...] = pe

    @pl.when(i == 0)
    def _():
        stats[...] = jnp.zeros((8, H), jnp.float32)

    stats[...] += _stats_update(pe)


def _node_mlp(xlo, xhi, alo, ahi, cnt, eps, p):
    n = xlo.shape[0]
    w1 = p["l1"]["W"]
    out, stats = pl.pallas_call(
        _node_mlp_body,
        grid=(n // ROW_BLK,),
        in_specs=[pl.BlockSpec((ROW_BLK, HH), _row)] * 4 +
                 [pl.BlockSpec((ROW_BLK, 1), _row),
                  pl.BlockSpec((1, 1), _fixed),
                  pl.BlockSpec((HH, H), _fixed), pl.BlockSpec((HH, H), _fixed),
                  pl.BlockSpec((1, H), _fixed),
                  pl.BlockSpec((H, H), _fixed), pl.BlockSpec((1, H), _fixed)],
        out_specs=[pl.BlockSpec((ROW_BLK, H), _row),
                   pl.BlockSpec((8, H), _fixed)],
        out_shape=[jax.ShapeDtypeStruct((n, H), jnp.float32),
                   jax.ShapeDtypeStruct((8, H), jnp.float32)],
    )(xlo, xhi, alo, ahi, cnt, eps.reshape(1, 1),
      w1[:HH], w1[HH:], p["l1"]["b"].reshape(1, H),
      p["l2"]["W"], p["l2"]["b"].reshape(1, H))
    return out, stats


def _scale_relu_body(pe, st, olo, ohi):
    v = jax.nn.relu(pe[...] * st[0:1, :] + st[1:2, :])
    olo[...] = v[:, :HH]
    ohi[...] = v[:, HH:]


def _scale_relu(pe, s, t):
    n = pe.shape[0]
    st = jnp.stack([s, t])
    out = pl.pallas_call(
        _scale_relu_body,
        grid=(n // ROW_BLK,),
        in_specs=[pl.BlockSpec((ROW_BLK, H), _row),
                  pl.BlockSpec((2, H), _fixed)],
        out_specs=[pl.BlockSpec((ROW_BLK, HH), _row)] * 2,
        out_shape=[jax.ShapeDtypeStruct((n, HH), jnp.float32)] * 2,
    )(pe, st)
    return out


def _head_body(*refs):
    xs = refs[:10]
    w1s = refs[10:20]
    b1, w2, b2, w3, b3, w4, b4, out = refs[20:]
    acc = jnp.dot(xs[0][...], w1s[0][...], preferred_element_type=jnp.float32)
    for k in range(1, 10):
        acc += jnp.dot(xs[k][...], w1s[k][...], preferred_element_type=jnp.float32)
    h = jax.nn.relu(acc + b1[...])
    h = jax.nn.relu(jnp.dot(h, w2[...], preferred_element_type=jnp.float32) + b2[...])
    h = jax.nn.relu(jnp.dot(h, w3[...], preferred_element_type=jnp.float32) + b3[...])
    o = jnp.dot(h, w4[...], preferred_element_type=jnp.float32) + b4[...]
    o0 = o[:, 0:1]
    o1 = o[:, 1:2]
    m = jnp.maximum(o0, o1)
    lse = m + jnp.log(jnp.exp(o0 - m) + jnp.exp(o1 - m))
    out[...] = o - lse


def _head(x_pairs, p):
    n = x_pairs[0][0].shape[0]
    xs = [a for pair in x_pairs for a in pair]
    w1 = p["lin1"]["W"]
    w1s = [w1[i * HH:(i + 1) * HH] for i in range(10)]
    w4p = jnp.pad(p["lin4"]["W"], ((0, 0), (0, 126)))
    b4p = jnp.pad(p["lin4"]["b"], (0, 126)).reshape(1, 128)
    xspec = pl.BlockSpec((ROW_BLK, HH), _row)
    wspec = pl.BlockSpec((H, H), _fixed)
    bspec = pl.BlockSpec((1, H), _fixed)
    out = pl.pallas_call(
        _head_body,
        grid=(n // ROW_BLK,),
        in_specs=[xspec] * 10 + [pl.BlockSpec((HH, H), _fixed)] * 10 +
                 [bspec, wspec, bspec, wspec, bspec,
                  pl.BlockSpec((H, 128), _fixed), pl.BlockSpec((1, 128), _fixed)],
        out_specs=pl.BlockSpec((ROW_BLK, 128), _row),
        out_shape=jax.ShapeDtypeStruct((n, 128), jnp.float32),
    )(*xs, *w1s, p["lin1"]["b"].reshape(1, H),
      p["lin2"]["W"], p["lin2"]["b"].reshape(1, H),
      p["lin3"]["W"], p["lin3"]["b"].reshape(1, H), w4p, b4p)
    return out[:, :2]


def _gather_msg_aggregate(src_pair, p_pair, s, t, src_idx, dst_idx, n_dst):
    source = jnp.concatenate(src_pair, axis=1)
    pe = jnp.concatenate(p_pair, axis=1)
    msg = jax.nn.relu(jnp.take(source, src_idx, axis=0) + pe * s + t)
    agg = jax.ops.segment_sum(msg, dst_idx, num_segments=n_dst)
    return agg[:, :HH], agg[:, HH:]


def _degree(dst_idx, n_dst):
    cnt = jax.ops.segment_sum(jnp.ones(dst_idx.shape, jnp.float32), dst_idx,
                              num_segments=n_dst)
    return cnt.reshape(n_dst, 1)


def _bipartite_sc(src_pair, tgt_pair, p_pair, est, src_idx, dst_idx, cnt,
                  eps, p, n_dst):
    alo, ahi = _gather_msg_aggregate(src_pair, p_pair, est[0], est[1],
                                     src_idx, dst_idx, n_dst)
    pe, stats = _node_mlp(tgt_pair[0], tgt_pair[1], alo, ahi, cnt, eps,
                          p["mlp"])
    s, t = _bn_affine(stats, n_dst, p["mlp"]["g"], p["mlp"]["be"])
    return _scale_relu(pe, s, t)


def kernel(var_node_features, con_node_features, edge_features_var,
           edge_features_con, params, edge_index_var, edge_index_con,
           num_nodes_var, num_nodes_con):
    nv = var_node_features.shape[0]
    nc = con_node_features.shape[0]
    ne = edge_index_var.shape[1]

    xv = [_enc2(var_node_features, params["var_enc"])]
    xc = [_enc2(con_node_features, params["con_enc"])]

    ev = []
    ec = []
    for i in range(4):
        pv = params["layers_var"][i]
        plo, phi, stats = _edge_mlp(edge_features_var, pv["edge_enc"])
        ev.append(((plo, phi),
                   _bn_affine(stats, ne, pv["edge_enc"]["g"], pv["edge_enc"]["be"])))
        pc = params["layers_con"][i]
        plo, phi, stats = _edge_mlp(edge_features_con, pc["edge_enc"])
        ec.append(((plo, phi),
                   _bn_affine(stats, ne, pc["edge_enc"]["g"], pc["edge_enc"]["be"])))

    cnt_c = _degree(edge_index_var[1], nc)
    cnt_v = _degree(edge_index_con[1], nv)

    for i in range(4):
        pv = params["layers_var"][i]
        xc.append(_bipartite_sc(xv[-1], xc[-1], ev[i][0], ev[i][1],
                                edge_index_var[0], edge_index_var[1], cnt_c,
                                pv["eps"], pv, nc))
        pc = params["layers_con"][i]
        xv.append(_bipartite_sc(xc[-1], xv[-1], ec[i][0], ec[i][1],
                                edge_index_con[0], edge_index_con[1], cnt_v,
                                pc["eps"], pc, nv))

    out = _head(xv, params)
    dep = 0.0 * ((jnp.asarray(num_nodes_var) - nv) +
                 (jnp.asarray(num_nodes_con) - nc)).astype(out.dtype)
    return out + dep
